# baseline (speedup 1.0000x reference)
import functools

import jax
import jax.numpy as jnp
from jax import lax
from jax.experimental import pallas as pl
from jax.experimental.pallas import tpu as pltpu
from jax.experimental.pallas import tpu_sc as plsc

_L = 16


def _build_sc_call(B: int):
    info = plsc.get_sparse_core_info()
    nw = info.num_cores * info.num_subcores
    chunk = B // nw
    assert chunk % _L == 0 and chunk % 8 == 0

    mesh = plsc.VectorSubcoreMesh(core_axis_name="c", subcore_axis_name="s")
    out = jax.ShapeDtypeStruct((B,), jnp.float32)
    fvec = pltpu.VMEM((_L,), jnp.float32)

    @functools.partial(
        pl.kernel,
        out_type=[out, out, out],
        mesh=mesh,
        scratch_types=[
            pltpu.VMEM((chunk,), jnp.int32),
            fvec,
            fvec, fvec, fvec, fvec,
            pltpu.VMEM((chunk,), jnp.float32),
            pltpu.VMEM((chunk,), jnp.float32),
            pltpu.VMEM((chunk,), jnp.float32),
        ],
    )
    def sc_fn(x_hbm, e_hbm, w1_hbm, b1_hbm, w2_hbm, b2_hbm,
              emb_hbm, dense_hbm, diff_hbm,
              x_v, e_v, w1_v, b1_v, w2_v, b2_v,
              emb_v, dense_v, diff_v):
        wid = lax.axis_index("s") * info.num_cores + lax.axis_index("c")
        base = wid * chunk
        pltpu.sync_copy(x_hbm.at[pl.ds(base, chunk)], x_v)
        pltpu.sync_copy(e_hbm, e_v)
        pltpu.sync_copy(w1_hbm, w1_v)
        pltpu.sync_copy(b1_hbm, b1_v)
        pltpu.sync_copy(w2_hbm, w2_v)
        pltpu.sync_copy(b2_hbm, b2_v)

        table = e_v[...]
        w1 = w1_v[...]
        bb1 = b1_v[...]
        w2 = w2_v[...]
        bb2 = b2_v[...]

        for i in range(chunk // _L):
            sl = pl.ds(i * _L, _L)
            idx = x_v[sl]
            emb = table.at[idx].get(mode="promise_in_bounds")
            emb_o = emb * w1 + bb1
            dense_o = idx.astype(jnp.float32) * w2 + bb2
            emb_v[sl] = emb_o
            dense_v[sl] = dense_o
            diff_v[sl] = emb_o - dense_o

        pltpu.sync_copy(emb_v, emb_hbm.at[pl.ds(base, chunk)])
        pltpu.sync_copy(dense_v, dense_hbm.at[pl.ds(base, chunk)])
        pltpu.sync_copy(diff_v, diff_hbm.at[pl.ds(base, chunk)])

    return sc_fn


def kernel(x, E, W1, b1, W2, b2):
    B = x.shape[0]
    x_flat = x.reshape(B).astype(jnp.int32)
    p16 = [jnp.broadcast_to(a.reshape(-1)[:1], (_L,)).astype(jnp.float32)
           for a in (E, W1, b1, W2, b2)]
    emb_o, dense_o, diff = _build_sc_call(B)(x_flat, *p16)
    return (emb_o.reshape(B, 1), dense_o.reshape(B, 1), diff.reshape(B, 1))


# speedup vs baseline: 1.1942x; 1.1942x over previous
import functools

import jax
import jax.numpy as jnp
from jax import lax
from jax.experimental import pallas as pl
from jax.experimental.pallas import tpu as pltpu
from jax.experimental.pallas import tpu_sc as plsc

_L = 16


def _build_sc_call(B: int):
    info = plsc.get_sparse_core_info()
    nw = info.num_cores * info.num_subcores
    chunk = B // nw
    assert chunk % _L == 0 and chunk % 8 == 0

    mesh = plsc.VectorSubcoreMesh(core_axis_name="c", subcore_axis_name="s")
    out = jax.ShapeDtypeStruct((B,), jnp.float32)

    @functools.partial(
        pl.kernel,
        out_type=[out, out, out],
        mesh=mesh,
        scratch_types=[
            pltpu.VMEM((chunk,), jnp.int32),
            pltpu.VMEM((_L,), jnp.float32),
            pltpu.VMEM((chunk,), jnp.float32),
            pltpu.VMEM((chunk,), jnp.float32),
            pltpu.VMEM((chunk,), jnp.float32),
            pltpu.SemaphoreType.DMA,
            pltpu.SemaphoreType.DMA,
        ],
    )
    def sc_fn(x_hbm, p_hbm,
              emb_hbm, dense_hbm, diff_hbm,
              x_v, p_v, emb_v, dense_v, diff_v,
              in_sem, out_sem):
        wid = lax.axis_index("s") * info.num_cores + lax.axis_index("c")
        base = wid * chunk
        cp_x = pltpu.async_copy(x_hbm.at[pl.ds(base, chunk)], x_v, in_sem)
        cp_p = pltpu.async_copy(p_hbm, p_v, in_sem)
        cp_x.wait()
        cp_p.wait()

        p = p_v[...]
        lane = lambda k: p.at[jnp.full((_L,), k, jnp.int32)].get(
            mode="promise_in_bounds")
        w1, bb1, w2, bb2 = lane(1), lane(2), lane(3), lane(4)

        for i in range(chunk // _L):
            sl = pl.ds(i * _L, _L)
            idx = x_v[sl]
            emb = p.at[idx].get(mode="promise_in_bounds")
            emb_o = emb * w1 + bb1
            dense_o = idx.astype(jnp.float32) * w2 + bb2
            emb_v[sl] = emb_o
            dense_v[sl] = dense_o
            diff_v[sl] = emb_o - dense_o

        cp_e = pltpu.async_copy(emb_v, emb_hbm.at[pl.ds(base, chunk)], out_sem)
        cp_d = pltpu.async_copy(dense_v, dense_hbm.at[pl.ds(base, chunk)], out_sem)
        cp_f = pltpu.async_copy(diff_v, diff_hbm.at[pl.ds(base, chunk)], out_sem)
        cp_e.wait()
        cp_d.wait()
        cp_f.wait()

    return sc_fn


def kernel(x, E, W1, b1, W2, b2):
    B = x.shape[0]
    x_flat = x.reshape(B).astype(jnp.int32)
    packed = jnp.concatenate(
        [a.reshape(-1)[:1] for a in (E, W1, b1, W2, b2)]
        + [jnp.zeros((_L - 5,), jnp.float32)]).astype(jnp.float32)
    emb_o, dense_o, diff = _build_sc_call(B)(x_flat, packed)
    return (emb_o.reshape(B, 1), dense_o.reshape(B, 1), diff.reshape(B, 1))


# speedup vs baseline: 1.3363x; 1.1190x over previous
import functools

import jax
import jax.numpy as jnp
from jax import lax
from jax.experimental import pallas as pl
from jax.experimental.pallas import tpu as pltpu
from jax.experimental.pallas import tpu_sc as plsc

_L = 16


def _build_sc_call(B: int):
    info = plsc.get_sparse_core_info()
    nc = 1
    nw = nc * info.num_subcores
    chunk = B // nw
    assert chunk % _L == 0 and chunk % 8 == 0

    mesh = plsc.VectorSubcoreMesh(
        core_axis_name="c", subcore_axis_name="s", num_cores=nc)
    out = jax.ShapeDtypeStruct((B,), jnp.float32)

    @functools.partial(
        pl.kernel,
        out_type=[out, out, out],
        mesh=mesh,
        scratch_types=[
            pltpu.VMEM((chunk,), jnp.int32),
            pltpu.VMEM((_L,), jnp.float32),
            pltpu.VMEM((chunk,), jnp.float32),
            pltpu.VMEM((chunk,), jnp.float32),
            pltpu.VMEM((chunk,), jnp.float32),
            pltpu.SemaphoreType.DMA,
            pltpu.SemaphoreType.DMA,
        ],
    )
    def sc_fn(x_hbm, p_hbm,
              emb_hbm, dense_hbm, diff_hbm,
              x_v, p_v, emb_v, dense_v, diff_v,
              in_sem, out_sem):
        wid = lax.axis_index("s") * nc + lax.axis_index("c")
        base = wid * chunk
        cp_x = pltpu.async_copy(x_hbm.at[pl.ds(base, chunk)], x_v, in_sem)
        cp_p = pltpu.async_copy(p_hbm, p_v, in_sem)
        cp_x.wait()
        cp_p.wait()

        p = p_v[...]
        lane = lambda k: p.at[jnp.full((_L,), k, jnp.int32)].get(
            mode="promise_in_bounds")
        w1, bb1, w2, bb2 = lane(1), lane(2), lane(3), lane(4)

        for i in range(chunk // _L):
            sl = pl.ds(i * _L, _L)
            idx = x_v[sl]
            emb = p.at[idx].get(mode="promise_in_bounds")
            emb_o = emb * w1 + bb1
            dense_o = idx.astype(jnp.float32) * w2 + bb2
            emb_v[sl] = emb_o
            dense_v[sl] = dense_o
            diff_v[sl] = emb_o - dense_o

        cp_e = pltpu.async_copy(emb_v, emb_hbm.at[pl.ds(base, chunk)], out_sem)
        cp_d = pltpu.async_copy(dense_v, dense_hbm.at[pl.ds(base, chunk)], out_sem)
        cp_f = pltpu.async_copy(diff_v, diff_hbm.at[pl.ds(base, chunk)], out_sem)
        cp_e.wait()
        cp_d.wait()
        cp_f.wait()

    return sc_fn


def kernel(x, E, W1, b1, W2, b2):
    B = x.shape[0]
    x_flat = x.reshape(B).astype(jnp.int32)
    packed = jnp.concatenate(
        [a.reshape(-1)[:1] for a in (E, W1, b1, W2, b2)]
        + [jnp.zeros((_L - 5,), jnp.float32)]).astype(jnp.float32)
    emb_o, dense_o, diff = _build_sc_call(B)(x_flat, packed)
    return (emb_o.reshape(B, 1), dense_o.reshape(B, 1), diff.reshape(B, 1))
